# baseline (speedup 1.0000x reference)
import functools

import jax
import jax.numpy as jnp
from jax import lax
from jax.experimental import pallas as pl
from jax.experimental.pallas import tpu as pltpu
from jax.experimental.pallas import tpu_sc as plsc

NC = 2
NS = 16
NW = NC * NS
K = 128
KG = 64


def _cdiv(a, b):
    return (a + b - 1) // b


def _stage_blocks(rpt, blk=K):
    return [(r * blk, min(blk, rpt - r * blk)) for r in range(_cdiv(rpt, blk))]


SLOW_FRAC = 0.34


@functools.lru_cache(maxsize=None)
def _sc_aggregate(N, D, EP, NP):
    CH = EP // (NW * KG)
    CH0 = 2 * int(round(CH * SLOW_FRAC / 2))
    CH1 = 2 * CH - CH0
    RPT = NP // NS

    mesh = plsc.VectorSubcoreMesh(core_axis_name="c", subcore_axis_name="s")

    @functools.partial(
        pl.kernel,
        mesh=mesh,
        out_type=jax.ShapeDtypeStruct((NC * NP, D), jnp.float32),
        scratch_types=[
            pltpu.VMEM((2 * KG,), jnp.int32),
            pltpu.VMEM((2 * KG,), jnp.int32),
            pltpu.VMEM((KG,), jnp.int32),
            pltpu.VMEM((KG,), jnp.int32),
            pltpu.VMEM((KG,), jnp.int32),
            pltpu.VMEM((KG,), jnp.int32),
            pltpu.VMEM((KG, D), jnp.float32),
            pltpu.VMEM((KG, D), jnp.float32),
            pltpu.VMEM_SHARED((NP, D), jnp.float32),
            pltpu.SemaphoreType.DMA,
            pltpu.SemaphoreType.DMA,
            pltpu.SemaphoreType.DMA,
            pltpu.SemaphoreType.DMA,
            pltpu.SemaphoreType.DMA,
            pltpu.SemaphoreType.DMA,
        ],
    )
    def agg(x_hbm, pairs_hbm, zn_hbm, nei_out,
            p0, p1, s0, s1, d0, d1, r0, r1, acc,
            ps0, ps1, gs0, gs1, ss0, ss1):
        cid = lax.axis_index("c")
        sid = lax.axis_index("s")
        wid = cid * NS + sid
        pairs = (p0, p1)
        srcs = (s0, s1)
        dsts = (d0, d1)
        rows = (r0, r1)
        psems = (ps0, ps1)
        gsems = (gs0, gs1)
        ssems = (ss0, ss1)

        rows0 = sid * RPT
        pltpu.sync_copy(zn_hbm, r0)
        for off, sz in _stage_blocks(RPT, KG):
            pltpu.sync_copy(r0.at[pl.ds(0, sz)],
                            acc.at[pl.ds(rows0 + off, sz)])
        plsc.subcore_barrier()

        chT = jnp.where(cid == 0, CH0, CH1)
        prow0 = jnp.where(cid == 0, sid * CH0, NS * CH0 + sid * CH1)
        pltpu.async_copy(pairs_hbm.at[prow0], p0, ps0)
        pltpu.async_copy(pairs_hbm.at[prow0 + 1], p1, ps1)

        def pair_iter(g2, carry):
            for b in (0, 1):
                g = g2 * 2 + b
                nb = 1 - b
                pb, sb, db, rb = pairs[b], srcs[b], dsts[b], rows[b]

                pltpu.make_async_copy(pairs_hbm.at[prow0], pb,
                                      psems[b]).wait()

                @pl.when(g >= 2)
                def _wait_prev_scatter():
                    pltpu.make_async_copy(rb, acc.at[db], ssems[b]).wait()

                for i in range(KG // 16):
                    sb[pl.ds(i * 16, 16)] = pb[pl.ds(i * 16, 16)]
                    db[pl.ds(i * 16, 16)] = pb[pl.ds(KG + i * 16, 16)]

                @pl.when(g + 2 < chT)
                def _prefetch_pair():
                    pltpu.async_copy(pairs_hbm.at[prow0 + g + 2], pb,
                                     psems[b])

                pltpu.async_copy(x_hbm.at[sb], rb, gsems[b])

                @pl.when(g >= 1)
                def _scatter_prev():
                    pltpu.make_async_copy(x_hbm.at[srcs[nb]], rows[nb],
                                          gsems[nb]).wait()
                    pltpu.async_copy(rows[nb], acc.at[dsts[nb]],
                                     ssems[nb], add=True)
            return carry

        lax.fori_loop(0, chT // 2, pair_iter, 0)
        pltpu.make_async_copy(x_hbm.at[s1], r1, gs1).wait()
        pltpu.async_copy(r1, acc.at[d1], ss1, add=True)
        pltpu.make_async_copy(r0, acc.at[d0], ss0).wait()
        pltpu.make_async_copy(r1, acc.at[d1], ss1).wait()
        plsc.subcore_barrier()

        out0 = cid * NP + rows0
        for off, sz in _stage_blocks(RPT, KG):
            buf = rows[(off // KG) % 2]
            pltpu.sync_copy(acc.at[pl.ds(rows0 + off, sz)],
                            buf.at[pl.ds(0, sz)])
            pltpu.sync_copy(buf.at[pl.ds(0, sz)],
                            nei_out.at[pl.ds(out0 + off, sz)])

    return agg


@functools.lru_cache(maxsize=None)
def _sc_degree(D, EP, NP):
    CH = EP // (NW * K)
    RPT = NP // NS

    mesh = plsc.VectorSubcoreMesh(core_axis_name="c", subcore_axis_name="s")

    @functools.partial(
        pl.kernel,
        mesh=mesh,
        out_type=jax.ShapeDtypeStruct((NC * NP, D), jnp.float32),
        scratch_types=[
            pltpu.VMEM((K,), jnp.int32),
            pltpu.VMEM((K, D), jnp.float32),
            pltpu.VMEM_SHARED((NP, D), jnp.float32),
        ],
    )
    def deg(dstp_hbm, zn_hbm, ones_hbm, deg_out,
            dst_v, st_v, dacc):
        cid = lax.axis_index("c")
        sid = lax.axis_index("s")
        wid = cid * NS + sid

        rows0 = sid * RPT
        pltpu.sync_copy(zn_hbm, st_v)
        for off, sz in _stage_blocks(RPT):
            pltpu.sync_copy(st_v.at[pl.ds(0, sz)],
                            dacc.at[pl.ds(rows0 + off, sz)])
        pltpu.sync_copy(ones_hbm, st_v)
        plsc.subcore_barrier()

        def chunk(g, carry):
            base = (wid * CH + g) * K
            pltpu.sync_copy(dstp_hbm.at[pl.ds(base, K)], dst_v)
            pltpu.sync_copy(st_v, dacc.at[dst_v], add=True)
            return carry

        lax.fori_loop(0, CH, chunk, 0)
        plsc.subcore_barrier()

        out0 = cid * NP + rows0
        for off, sz in _stage_blocks(RPT):
            pltpu.sync_copy(dacc.at[pl.ds(rows0 + off, sz)],
                            st_v.at[pl.ds(0, sz)])
            pltpu.sync_copy(st_v.at[pl.ds(0, sz)],
                            deg_out.at[pl.ds(out0 + off, sz)])

    return deg


def _finalize_body(x_ref, nei_ref, deg_ref, w_ref, b_ref, o_ref, *, D):
    nei = nei_ref[0] + nei_ref[1]
    deg = deg_ref[0, :, 0:1] + deg_ref[1, :, 0:1]
    mean = nei / jnp.maximum(deg, 1.0)
    o_ref[...] = (
        jnp.dot(x_ref[...], w_ref[0:D, :], preferred_element_type=jnp.float32)
        + jnp.dot(mean, w_ref[D:, :], preferred_element_type=jnp.float32)
        + b_ref[...]
    )


def kernel(x, edge_index, weight, bias):
    N, D = x.shape
    E = edge_index.shape[1]
    OUT = weight.shape[1]

    CH = _cdiv(E, NW * K)
    EP = NW * K * CH
    NP = _cdiv(N + 1, 128) * 128

    src = edge_index[0]
    dst = edge_index[1]
    pad = EP - E
    srcp = jnp.concatenate([src, jnp.zeros((pad,), jnp.int32)])
    dstp = jnp.concatenate([dst, jnp.full((pad,), N, jnp.int32)])
    pairs = jnp.concatenate(
        [srcp.reshape(-1, KG), dstp.reshape(-1, KG)], axis=1)
    zn = jnp.zeros((K, D), jnp.float32)
    zn_g = jnp.zeros((KG, D), jnp.float32)
    ones_k = jnp.ones((K, D), jnp.float32)

    nei_flat = _sc_aggregate(N, D, EP, NP)(x, pairs, zn_g)
    deg_flat = _sc_degree(D, EP, NP)(dstp, zn, ones_k)
    nei_p = nei_flat.reshape(NC, NP, D)
    deg_p = deg_flat.reshape(NC, NP, D)

    BR = 2000
    out = pl.pallas_call(
        functools.partial(_finalize_body, D=D),
        grid=(N // BR,),
        in_specs=[
            pl.BlockSpec((BR, D), lambda i: (i, 0)),
            pl.BlockSpec((NC, BR, D), lambda i: (0, i, 0)),
            pl.BlockSpec((NC, BR, D), lambda i: (0, i, 0)),
            pl.BlockSpec((2 * D, OUT), lambda i: (0, 0)),
            pl.BlockSpec((1, OUT), lambda i: (0, 0)),
        ],
        out_specs=pl.BlockSpec((BR, OUT), lambda i: (i, 0)),
        out_shape=jax.ShapeDtypeStruct((N, OUT), jnp.float32),
    )(x, nei_p, deg_p, weight, bias.reshape(1, OUT))
    return out


# speedup vs baseline: 1.3422x; 1.3422x over previous
import functools

import jax
import jax.numpy as jnp
from jax import lax
from jax.experimental import pallas as pl
from jax.experimental.pallas import tpu as pltpu
from jax.experimental.pallas import tpu_sc as plsc

NC = 2
NS = 16
NW = NC * NS
K = 128
KG = 64


def _cdiv(a, b):
    return (a + b - 1) // b


def _stage_blocks(rpt, blk=K):
    return [(r * blk, min(blk, rpt - r * blk)) for r in range(_cdiv(rpt, blk))]


SLOW_FRAC = 0.34


@functools.lru_cache(maxsize=None)
def _sc_aggregate(N, D, EP, NP):
    CH = EP // (NW * KG)
    CH1 = 2 * int(round(CH * SLOW_FRAC / 2))
    CH0 = 2 * CH - CH1
    RPT = NP // NS

    mesh = plsc.VectorSubcoreMesh(core_axis_name="c", subcore_axis_name="s")

    @functools.partial(
        pl.kernel,
        mesh=mesh,
        out_type=jax.ShapeDtypeStruct((NC * NP, D), jnp.float32),
        scratch_types=[
            pltpu.VMEM((2 * KG,), jnp.int32),
            pltpu.VMEM((2 * KG,), jnp.int32),
            pltpu.VMEM((KG,), jnp.int32),
            pltpu.VMEM((KG,), jnp.int32),
            pltpu.VMEM((KG,), jnp.int32),
            pltpu.VMEM((KG,), jnp.int32),
            pltpu.VMEM((KG, D), jnp.float32),
            pltpu.VMEM((KG, D), jnp.float32),
            pltpu.VMEM_SHARED((NP, D), jnp.float32),
            pltpu.SemaphoreType.DMA,
            pltpu.SemaphoreType.DMA,
            pltpu.SemaphoreType.DMA,
            pltpu.SemaphoreType.DMA,
            pltpu.SemaphoreType.DMA,
            pltpu.SemaphoreType.DMA,
        ],
    )
    def agg(x_hbm, pairs_hbm, zn_hbm, nei_out,
            p0, p1, s0, s1, d0, d1, r0, r1, acc,
            ps0, ps1, gs0, gs1, ss0, ss1):
        cid = lax.axis_index("c")
        sid = lax.axis_index("s")
        wid = cid * NS + sid
        pairs = (p0, p1)
        srcs = (s0, s1)
        dsts = (d0, d1)
        rows = (r0, r1)
        psems = (ps0, ps1)
        gsems = (gs0, gs1)
        ssems = (ss0, ss1)

        rows0 = sid * RPT
        pltpu.sync_copy(zn_hbm, r0)
        for off, sz in _stage_blocks(RPT, KG):
            pltpu.sync_copy(r0.at[pl.ds(0, sz)],
                            acc.at[pl.ds(rows0 + off, sz)])
        plsc.subcore_barrier()

        chT = jnp.where(cid == 0, CH0, CH1)
        prow0 = jnp.where(cid == 0, sid * CH0, NS * CH0 + sid * CH1)
        pltpu.async_copy(pairs_hbm.at[prow0], p0, ps0)
        pltpu.async_copy(pairs_hbm.at[prow0 + 1], p1, ps1)

        def pair_iter(g2, carry):
            for b in (0, 1):
                g = g2 * 2 + b
                nb = 1 - b
                pb, sb, db, rb = pairs[b], srcs[b], dsts[b], rows[b]

                pltpu.make_async_copy(pairs_hbm.at[prow0], pb,
                                      psems[b]).wait()

                @pl.when(g >= 2)
                def _wait_prev_scatter():
                    pltpu.make_async_copy(rb, acc.at[db], ssems[b]).wait()

                for i in range(KG // 16):
                    sb[pl.ds(i * 16, 16)] = pb[pl.ds(i * 16, 16)]
                    db[pl.ds(i * 16, 16)] = pb[pl.ds(KG + i * 16, 16)]

                @pl.when(g + 2 < chT)
                def _prefetch_pair():
                    pltpu.async_copy(pairs_hbm.at[prow0 + g + 2], pb,
                                     psems[b])

                pltpu.async_copy(x_hbm.at[sb], rb, gsems[b])

                @pl.when(g >= 1)
                def _scatter_prev():
                    pltpu.make_async_copy(x_hbm.at[srcs[nb]], rows[nb],
                                          gsems[nb]).wait()
                    pltpu.async_copy(rows[nb], acc.at[dsts[nb]],
                                     ssems[nb], add=True)
            return carry

        lax.fori_loop(0, chT // 2, pair_iter, 0)
        pltpu.make_async_copy(x_hbm.at[s1], r1, gs1).wait()
        pltpu.async_copy(r1, acc.at[d1], ss1, add=True)
        pltpu.make_async_copy(r0, acc.at[d0], ss0).wait()
        pltpu.make_async_copy(r1, acc.at[d1], ss1).wait()
        plsc.subcore_barrier()

        out0 = cid * NP + rows0
        for off, sz in _stage_blocks(RPT, KG):
            buf = rows[(off // KG) % 2]
            pltpu.sync_copy(acc.at[pl.ds(rows0 + off, sz)],
                            buf.at[pl.ds(0, sz)])
            pltpu.sync_copy(buf.at[pl.ds(0, sz)],
                            nei_out.at[pl.ds(out0 + off, sz)])

    return agg


@functools.lru_cache(maxsize=None)
def _sc_degree(D, EP, NP):
    CH = EP // (NW * K)
    RPT = NP // NS

    mesh = plsc.VectorSubcoreMesh(core_axis_name="c", subcore_axis_name="s")

    @functools.partial(
        pl.kernel,
        mesh=mesh,
        out_type=jax.ShapeDtypeStruct((NC * NP, D), jnp.float32),
        scratch_types=[
            pltpu.VMEM((K,), jnp.int32),
            pltpu.VMEM((K, D), jnp.float32),
            pltpu.VMEM_SHARED((NP, D), jnp.float32),
        ],
    )
    def deg(dstp_hbm, zn_hbm, ones_hbm, deg_out,
            dst_v, st_v, dacc):
        cid = lax.axis_index("c")
        sid = lax.axis_index("s")
        wid = cid * NS + sid

        rows0 = sid * RPT
        pltpu.sync_copy(zn_hbm, st_v)
        for off, sz in _stage_blocks(RPT):
            pltpu.sync_copy(st_v.at[pl.ds(0, sz)],
                            dacc.at[pl.ds(rows0 + off, sz)])
        pltpu.sync_copy(ones_hbm, st_v)
        plsc.subcore_barrier()

        def chunk(g, carry):
            base = (wid * CH + g) * K
            pltpu.sync_copy(dstp_hbm.at[pl.ds(base, K)], dst_v)
            pltpu.sync_copy(st_v, dacc.at[dst_v], add=True)
            return carry

        lax.fori_loop(0, CH, chunk, 0)
        plsc.subcore_barrier()

        out0 = cid * NP + rows0
        for off, sz in _stage_blocks(RPT):
            pltpu.sync_copy(dacc.at[pl.ds(rows0 + off, sz)],
                            st_v.at[pl.ds(0, sz)])
            pltpu.sync_copy(st_v.at[pl.ds(0, sz)],
                            deg_out.at[pl.ds(out0 + off, sz)])

    return deg


def _finalize_body(x_ref, nei_ref, deg_ref, w_ref, b_ref, o_ref, *, D):
    nei = nei_ref[0] + nei_ref[1]
    deg = deg_ref[0, :, 0:1] + deg_ref[1, :, 0:1]
    mean = nei / jnp.maximum(deg, 1.0)
    o_ref[...] = (
        jnp.dot(x_ref[...], w_ref[0:D, :], preferred_element_type=jnp.float32)
        + jnp.dot(mean, w_ref[D:, :], preferred_element_type=jnp.float32)
        + b_ref[...]
    )


def kernel(x, edge_index, weight, bias):
    N, D = x.shape
    E = edge_index.shape[1]
    OUT = weight.shape[1]

    CH = _cdiv(E, NW * K)
    EP = NW * K * CH
    NP = _cdiv(N + 1, 128) * 128

    src = edge_index[0]
    dst = edge_index[1]
    pad = EP - E
    srcp = jnp.concatenate([src, jnp.zeros((pad,), jnp.int32)])
    dstp = jnp.concatenate([dst, jnp.full((pad,), N, jnp.int32)])
    pairs = jnp.concatenate(
        [srcp.reshape(-1, KG), dstp.reshape(-1, KG)], axis=1)
    zn = jnp.zeros((K, D), jnp.float32)
    zn_g = jnp.zeros((KG, D), jnp.float32)
    ones_k = jnp.ones((K, D), jnp.float32)

    nei_flat = _sc_aggregate(N, D, EP, NP)(x, pairs, zn_g)
    deg_flat = _sc_degree(D, EP, NP)(dstp, zn, ones_k)
    nei_p = nei_flat.reshape(NC, NP, D)
    deg_p = deg_flat.reshape(NC, NP, D)

    BR = 2000
    out = pl.pallas_call(
        functools.partial(_finalize_body, D=D),
        grid=(N // BR,),
        in_specs=[
            pl.BlockSpec((BR, D), lambda i: (i, 0)),
            pl.BlockSpec((NC, BR, D), lambda i: (0, i, 0)),
            pl.BlockSpec((NC, BR, D), lambda i: (0, i, 0)),
            pl.BlockSpec((2 * D, OUT), lambda i: (0, 0)),
            pl.BlockSpec((1, OUT), lambda i: (0, 0)),
        ],
        out_specs=pl.BlockSpec((BR, OUT), lambda i: (i, 0)),
        out_shape=jax.ShapeDtypeStruct((N, OUT), jnp.float32),
    )(x, nei_p, deg_p, weight, bias.reshape(1, OUT))
    return out
